# baseline (speedup 1.0000x reference)
import dataclasses
import functools

import jax
import jax.numpy as jnp
from jax import lax
from jax.experimental import pallas as pl
from jax.experimental.pallas import tpu as pltpu
from jax.experimental.pallas import tpu_sc as plsc

N = 320000
D = 128
N_SUB = 10000
N_SUB_PAD = 10240

BR = 2560
N_BLOCKS = N // BR
RB = BR // D


def _scores_kernel(f_ref, w_ref, eye_ref, e_ref, m_ref, *, nblocks):
    f = f_ref[...]
    fh = f.astype(jnp.bfloat16)
    fl = (f - fh.astype(jnp.float32)).astype(jnp.bfloat16)
    fhl = jnp.concatenate([fh, fl], axis=1)
    s2 = jax.lax.dot_general(fhl, w_ref[...], (((1,), (0,)), ((), ())),
                             preferred_element_type=jnp.float32)
    s3 = s2.reshape(RB, D, D) * eye_ref[...][None]
    t = jnp.sum(s3, axis=1)
    e = jnp.exp(t)
    e_ref[...] = e.reshape(1, RB, D)
    m_ref[...] = jnp.max(e, axis=1, keepdims=True).reshape(1, 1, RB)


def _make_scores(nblocks, blk0):
    def run(feats, whl, eye):
        e3d, m = pl.pallas_call(
            functools.partial(_scores_kernel, nblocks=nblocks),
            grid=(nblocks,),
            in_specs=[
                pl.BlockSpec((BR, D), lambda i: (i + blk0, 0)),
                pl.BlockSpec((2 * D, D), lambda i: (0, 0)),
                pl.BlockSpec((D, D), lambda i: (0, 0)),
            ],
            out_specs=[
                pl.BlockSpec((1, RB, D), lambda i: (i, 0, 0)),
                pl.BlockSpec((1, 1, RB), lambda i: (i, 0, 0)),
            ],
            out_shape=[
                jax.ShapeDtypeStruct((nblocks, RB, D), jnp.float32),
                jax.ShapeDtypeStruct((nblocks, 1, RB), jnp.float32),
            ],
        )(feats, whl, eye)
        return e3d.reshape(nblocks * BR), m
    return run


NC, NS = 2, 16
NW = NC * NS
CHUNK = 80
SPLIT_BLOCKS = (42, 42, 41)
SEG_PER_TILE = N_SUB_PAD // NS
ZROWS = N_SUB_PAD // D

_sc_mesh = plsc.VectorSubcoreMesh(core_axis_name="c", subcore_axis_name="s")

_sc_params = pltpu.CompilerParams()
if "needs_layout_passes" in pltpu.CompilerParams.__dataclass_fields__:
    _sc_params = dataclasses.replace(_sc_params, needs_layout_passes=False)


def _sc_body(f_hbm, e_hbm, id_hbm, z_hbm, out_hbm, zout_hbm,
             fbuf, ebuf, idxbuf, zbuf, acc, sin, ssc, *, row0, n_chunks):
    rows_per_tile = n_chunks * CHUNK
    c = lax.axis_index("c")
    s = lax.axis_index("s")
    wid = c * NS + s
    base = wid * rows_per_tile

    pltpu.sync_copy(z_hbm.at[pl.ds(s * SEG_PER_TILE, SEG_PER_TILE)],
                    acc.at[pl.ds(s * SEG_PER_TILE, SEG_PER_TILE)])
    pltpu.sync_copy(z_hbm.at[pl.ds(0, ZROWS)], zbuf)
    plsc.subcore_barrier()

    lanes = lax.broadcasted_iota(jnp.int32, (16,), 0)
    lanes_p1 = jnp.minimum(lanes + 1, 15)
    lanes_m1 = jnp.maximum(lanes - 1, 0)

    def in_descs(b, ci):
        off = base + ci * CHUNK
        return (
            pltpu.make_async_copy(f_hbm.at[pl.ds(row0 + off, CHUNK)],
                                  fbuf.at[b], sin.at[b]),
            pltpu.make_async_copy(e_hbm.at[pl.ds(off, CHUNK)], ebuf.at[b],
                                  sin.at[b]),
            pltpu.make_async_copy(id_hbm.at[pl.ds(row0 + off, CHUNK)],
                                  idxbuf.at[b], sin.at[b]),
        )

    def issue_in(b, ci):
        for d in in_descs(b, ci):
            d.start()

    def wait_in(b, ci):
        for d in in_descs(b, ci):
            d.wait()

    def sc_desc(b):
        return pltpu.make_async_copy(fbuf.at[b], acc.at[idxbuf.at[b]],
                                     ssc.at[b])

    def compute(b):
        @pl.loop(0, CHUNK // 16)
        def _g(g):
            go = g * 16
            ev = ebuf[b, pl.ds(go, 16)]
            for j in range(16):
                es = jnp.broadcast_to(ev[j], (16,))
                for v in range(D // 16):
                    fbuf[b, go + j, pl.ds(v * 16, 16)] = (
                        fbuf[b, go + j, pl.ds(v * 16, 16)] * es)

        @pl.loop(0, CHUNK // 16)
        def _z(g):
            go = g * 16
            ev = ebuf[b, pl.ds(go, 16)]
            idv = idxbuf[b, pl.ds(go, 16)]
            cs = plsc.cumsum(ev)
            id_next = idv.at[lanes_p1].get(mode="promise_in_bounds")
            id_prev = idv.at[lanes_m1].get(mode="promise_in_bounds")
            is_end = (lanes == 15) | (idv != id_next)
            is_start = (lanes == 0) | (idv != id_prev)
            start_lane = plsc.cummax(jnp.where(is_start, lanes, 0))
            cs_before = cs.at[jnp.maximum(start_lane - 1, 0)].get(
                mode="promise_in_bounds")
            run_sum = cs - jnp.where(start_lane > 0, cs_before, 0.0)
            plsc.addupdate_scatter(
                zbuf,
                [lax.shift_right_logical(idv, 7),
                 lax.bitwise_and(idv, 127)],
                run_sum, mask=is_end)

    def position(ci, b, first=False):
        wait_in(b, ci)
        compute(b)
        pltpu.async_copy(fbuf.at[b], acc.at[idxbuf.at[b]], ssc.at[b],
                         add=True)
        if not first:
            sc_desc((b - 1) % 3).wait()
        nxt = (b + 2) % 3
        if isinstance(ci, int):
            if ci + 2 < n_chunks:
                issue_in(nxt, ci + 2)
        else:
            @pl.when(ci + 2 < n_chunks)
            def _():
                issue_in(nxt, ci + 2)

    issue_in(0, 0)
    issue_in(1, 1)
    position(0, 0, first=True)
    position(1, 1)

    n_loop = (n_chunks - 2) // 3
    n_rem = (n_chunks - 2) % 3

    @pl.loop(0, n_loop)
    def _k(k):
        p = 2 + 3 * k
        position(p, 2)
        position(p + 1, 0)
        position(p + 2, 1)

    for t in range(n_rem):
        p = 2 + 3 * n_loop + t
        position(p, p % 3)

    sc_desc((n_chunks - 1) % 3).wait()
    plsc.subcore_barrier()
    pltpu.sync_copy(acc.at[pl.ds(s * SEG_PER_TILE, SEG_PER_TILE)],
                    out_hbm.at[c, pl.ds(s * SEG_PER_TILE, SEG_PER_TILE)])
    pltpu.sync_copy(zbuf, zout_hbm.at[wid])


def _make_sc(row0, n_chunks):
    return pl.kernel(
        functools.partial(_sc_body, row0=row0, n_chunks=n_chunks),
        out_type=[
            jax.ShapeDtypeStruct((NC, N_SUB_PAD, D), jnp.float32),
            jax.ShapeDtypeStruct((NW, ZROWS, D), jnp.float32),
        ],
        mesh=_sc_mesh,
        scratch_types=[
            pltpu.VMEM((3, CHUNK, D), jnp.float32),
            pltpu.VMEM((3, CHUNK), jnp.float32),
            pltpu.VMEM((3, CHUNK), jnp.int32),
            pltpu.VMEM((ZROWS, D), jnp.float32),
            pltpu.VMEM_SHARED((N_SUB_PAD, D), jnp.float32),
            pltpu.SemaphoreType.DMA((3,)),
            pltpu.SemaphoreType.DMA((3,)),
        ],
        compiler_params=_sc_params,
    )


_scores_calls = []
_sc_calls = []
_blk0 = 0
for _nb in SPLIT_BLOCKS:
    _scores_calls.append(_make_scores(_nb, _blk0))
    _sc_calls.append(_make_sc(_blk0 * BR, _nb * BR // NW // CHUNK))
    _blk0 += _nb


BSEG = 1024
N_SEG_BLOCKS = N_SUB_PAD // BSEG
ZB = BSEG // D


def _combine_kernel(*refs):
    nq = len(SPLIT_BLOCKS)
    p_refs = refs[:nq]
    z_refs = refs[nq:2 * nq]
    m_ref = refs[2 * nq]
    o_ref = refs[2 * nq + 1]
    p = sum(r[0] + r[1] for r in p_refs)
    zsum = sum(jnp.sum(r[...], axis=0) for r in z_refs)
    eps = 1e-8 * m_ref[0, 0]
    p3 = p.reshape(ZB, D, D)
    o3 = p3 / (zsum[:, :, None] + eps)
    o_ref[...] = o3.reshape(BSEG, D)


def _combine(ps, zs, m):
    nq = len(SPLIT_BLOCKS)
    return pl.pallas_call(
        _combine_kernel,
        grid=(N_SEG_BLOCKS,),
        in_specs=(
            [pl.BlockSpec((NC, BSEG, D), lambda i: (0, i, 0))] * nq
            + [pl.BlockSpec((NW, ZB, D), lambda i: (0, i, 0))] * nq
            + [pl.BlockSpec(memory_space=pltpu.SMEM)]
        ),
        out_specs=pl.BlockSpec((BSEG, D), lambda i: (i, 0)),
        out_shape=jax.ShapeDtypeStruct((N_SUB_PAD, D), jnp.float32),
    )(*ps, *zs, m)


@jax.jit
def kernel(feats, subject_ids, W_att, b_att):
    del b_att
    ids = subject_ids.astype(jnp.int32)
    wh = W_att.astype(jnp.bfloat16)
    whl = jnp.broadcast_to(jnp.concatenate([wh, wh], axis=0), (2 * D, D))
    eye = jnp.eye(D, dtype=jnp.float32)
    zeros = jnp.zeros((N_SUB_PAD, D), jnp.float32)
    es = [sc(feats, whl, eye) for sc in _scores_calls]
    outs = [scq(feats, e, ids, zeros) for scq, (e, _) in zip(_sc_calls, es)]
    max_e = jnp.maximum(jnp.max(es[0][1]), jnp.max(es[1][1]))
    for _, mq in es[2:]:
        max_e = jnp.maximum(max_e, jnp.max(mq))
    m = max_e.reshape(1, 1)
    out_pad = _combine([p for p, _ in outs], [z for _, z in outs], m)
    return out_pad[:N_SUB]


# speedup vs baseline: 1.0192x; 1.0192x over previous
import dataclasses
import functools

import jax
import jax.numpy as jnp
from jax import lax
from jax.experimental import pallas as pl
from jax.experimental.pallas import tpu as pltpu
from jax.experimental.pallas import tpu_sc as plsc

N = 320000
D = 128
N_SUB = 10000
N_SUB_PAD = 10240

BR = 2560
N_BLOCKS = N // BR
RB = BR // D


def _scores_kernel(f_ref, w_ref, eye_ref, e_ref, m_ref, *, nblocks):
    f = f_ref[...]
    fh = f.astype(jnp.bfloat16)
    fl = (f - fh.astype(jnp.float32)).astype(jnp.bfloat16)
    fhl = jnp.concatenate([fh, fl], axis=1)
    s2 = jax.lax.dot_general(fhl, w_ref[...], (((1,), (0,)), ((), ())),
                             preferred_element_type=jnp.float32)
    s3 = s2.reshape(RB, D, D) * eye_ref[...][None]
    t = jnp.sum(s3, axis=1)
    e = jnp.exp(t)
    e_ref[...] = e.reshape(1, RB, D)
    m_ref[...] = jnp.max(e, axis=1, keepdims=True).reshape(1, 1, RB)


def _make_scores(nblocks, blk0):
    def run(feats, whl, eye):
        e3d, m = pl.pallas_call(
            functools.partial(_scores_kernel, nblocks=nblocks),
            grid=(nblocks,),
            in_specs=[
                pl.BlockSpec((BR, D), lambda i: (i + blk0, 0)),
                pl.BlockSpec((2 * D, D), lambda i: (0, 0)),
                pl.BlockSpec((D, D), lambda i: (0, 0)),
            ],
            out_specs=[
                pl.BlockSpec((1, RB, D), lambda i: (i, 0, 0)),
                pl.BlockSpec((1, 1, RB), lambda i: (i, 0, 0)),
            ],
            out_shape=[
                jax.ShapeDtypeStruct((nblocks, RB, D), jnp.float32),
                jax.ShapeDtypeStruct((nblocks, 1, RB), jnp.float32),
            ],
        )(feats, whl, eye)
        return e3d.reshape(nblocks * BR), m
    return run


NC, NS = 2, 16
NW = NC * NS
CHUNK = 80
SPLIT_BLOCKS = (42, 42, 41)
SEG_PER_TILE = N_SUB_PAD // NS
ZROWS = N_SUB_PAD // D

_sc_mesh = plsc.VectorSubcoreMesh(core_axis_name="c", subcore_axis_name="s")

_sc_params = pltpu.CompilerParams()
if "needs_layout_passes" in pltpu.CompilerParams.__dataclass_fields__:
    _sc_params = dataclasses.replace(_sc_params, needs_layout_passes=False)


def _sc_body(f_hbm, e_hbm, id_hbm, z_hbm, out_hbm, zout_hbm,
             fbuf, ebuf, idxbuf, zbuf, acc, sin, ssc, *, row0, n_chunks):
    rows_per_tile = n_chunks * CHUNK
    c = lax.axis_index("c")
    s = lax.axis_index("s")
    wid = c * NS + s
    base = wid * rows_per_tile

    pltpu.sync_copy(z_hbm.at[pl.ds(s * SEG_PER_TILE, SEG_PER_TILE)],
                    acc.at[pl.ds(s * SEG_PER_TILE, SEG_PER_TILE)])
    pltpu.sync_copy(z_hbm.at[pl.ds(0, ZROWS)], zbuf)
    plsc.subcore_barrier()

    lanes = lax.broadcasted_iota(jnp.int32, (16,), 0)
    lanes_p1 = jnp.minimum(lanes + 1, 15)
    lanes_m1 = jnp.maximum(lanes - 1, 0)

    def in_descs(b, ci):
        off = base + ci * CHUNK
        return (
            pltpu.make_async_copy(f_hbm.at[pl.ds(row0 + off, CHUNK)],
                                  fbuf.at[b], sin.at[b]),
            pltpu.make_async_copy(e_hbm.at[pl.ds(off, CHUNK)], ebuf.at[b],
                                  sin.at[b]),
            pltpu.make_async_copy(id_hbm.at[pl.ds(row0 + off, CHUNK)],
                                  idxbuf.at[b], sin.at[b]),
        )

    def issue_in(b, ci):
        for d in in_descs(b, ci):
            d.start()

    def wait_in(b, ci):
        for d in in_descs(b, ci):
            d.wait()

    def sc_desc(b):
        return pltpu.make_async_copy(fbuf.at[b], acc.at[idxbuf.at[b]],
                                     ssc.at[b])

    def compute(b):
        @pl.loop(0, CHUNK // 16)
        def _g(g):
            go = g * 16
            ev = ebuf[b, pl.ds(go, 16)]
            idv = idxbuf[b, pl.ds(go, 16)]
            for j in range(16):
                es = jnp.broadcast_to(ev[j], (16,))
                for v in range(D // 16):
                    fbuf[b, go + j, pl.ds(v * 16, 16)] = (
                        fbuf[b, go + j, pl.ds(v * 16, 16)] * es)
            cs = plsc.cumsum(ev)
            id_next = idv.at[lanes_p1].get(mode="promise_in_bounds")
            id_prev = idv.at[lanes_m1].get(mode="promise_in_bounds")
            is_end = (lanes == 15) | (idv != id_next)
            is_start = (lanes == 0) | (idv != id_prev)
            start_lane = plsc.cummax(jnp.where(is_start, lanes, 0))
            cs_before = cs.at[jnp.maximum(start_lane - 1, 0)].get(
                mode="promise_in_bounds")
            run_sum = cs - jnp.where(start_lane > 0, cs_before, 0.0)
            plsc.addupdate_scatter(
                zbuf,
                [lax.shift_right_logical(idv, 7),
                 lax.bitwise_and(idv, 127)],
                run_sum, mask=is_end)

    def position(ci, b, first=False):
        wait_in(b, ci)
        compute(b)
        pltpu.async_copy(fbuf.at[b], acc.at[idxbuf.at[b]], ssc.at[b],
                         add=True)
        if not first:
            sc_desc((b - 1) % 3).wait()
        nxt = (b + 2) % 3
        if isinstance(ci, int):
            if ci + 2 < n_chunks:
                issue_in(nxt, ci + 2)
        else:
            @pl.when(ci + 2 < n_chunks)
            def _():
                issue_in(nxt, ci + 2)

    issue_in(0, 0)
    issue_in(1, 1)
    position(0, 0, first=True)
    position(1, 1)

    n_loop = (n_chunks - 2) // 3
    n_rem = (n_chunks - 2) % 3

    @pl.loop(0, n_loop)
    def _k(k):
        p = 2 + 3 * k
        position(p, 2)
        position(p + 1, 0)
        position(p + 2, 1)

    for t in range(n_rem):
        p = 2 + 3 * n_loop + t
        position(p, p % 3)

    sc_desc((n_chunks - 1) % 3).wait()
    plsc.subcore_barrier()
    pltpu.sync_copy(acc.at[pl.ds(s * SEG_PER_TILE, SEG_PER_TILE)],
                    out_hbm.at[c, pl.ds(s * SEG_PER_TILE, SEG_PER_TILE)])
    pltpu.sync_copy(zbuf, zout_hbm.at[wid])


def _make_sc(row0, n_chunks):
    return pl.kernel(
        functools.partial(_sc_body, row0=row0, n_chunks=n_chunks),
        out_type=[
            jax.ShapeDtypeStruct((NC, N_SUB_PAD, D), jnp.float32),
            jax.ShapeDtypeStruct((NW, ZROWS, D), jnp.float32),
        ],
        mesh=_sc_mesh,
        scratch_types=[
            pltpu.VMEM((3, CHUNK, D), jnp.float32),
            pltpu.VMEM((3, CHUNK), jnp.float32),
            pltpu.VMEM((3, CHUNK), jnp.int32),
            pltpu.VMEM((ZROWS, D), jnp.float32),
            pltpu.VMEM_SHARED((N_SUB_PAD, D), jnp.float32),
            pltpu.SemaphoreType.DMA((3,)),
            pltpu.SemaphoreType.DMA((3,)),
        ],
        compiler_params=_sc_params,
    )


_scores_calls = []
_sc_calls = []
_blk0 = 0
for _nb in SPLIT_BLOCKS:
    _scores_calls.append(_make_scores(_nb, _blk0))
    _sc_calls.append(_make_sc(_blk0 * BR, _nb * BR // NW // CHUNK))
    _blk0 += _nb


BSEG = 1024
N_SEG_BLOCKS = N_SUB_PAD // BSEG
ZB = BSEG // D


def _combine_kernel(*refs):
    nq = len(SPLIT_BLOCKS)
    p_refs = refs[:nq]
    z_refs = refs[nq:2 * nq]
    m_ref = refs[2 * nq]
    o_ref = refs[2 * nq + 1]
    p = sum(r[0] + r[1] for r in p_refs)
    zsum = sum(jnp.sum(r[...], axis=0) for r in z_refs)
    eps = 1e-8 * m_ref[0, 0]
    p3 = p.reshape(ZB, D, D)
    o3 = p3 / (zsum[:, :, None] + eps)
    o_ref[...] = o3.reshape(BSEG, D)


def _combine(ps, zs, m):
    nq = len(SPLIT_BLOCKS)
    return pl.pallas_call(
        _combine_kernel,
        grid=(N_SEG_BLOCKS,),
        in_specs=(
            [pl.BlockSpec((NC, BSEG, D), lambda i: (0, i, 0))] * nq
            + [pl.BlockSpec((NW, ZB, D), lambda i: (0, i, 0))] * nq
            + [pl.BlockSpec(memory_space=pltpu.SMEM)]
        ),
        out_specs=pl.BlockSpec((BSEG, D), lambda i: (i, 0)),
        out_shape=jax.ShapeDtypeStruct((N_SUB_PAD, D), jnp.float32),
    )(*ps, *zs, m)


@jax.jit
def kernel(feats, subject_ids, W_att, b_att):
    del b_att
    ids = subject_ids.astype(jnp.int32)
    wh = W_att.astype(jnp.bfloat16)
    whl = jnp.broadcast_to(jnp.concatenate([wh, wh], axis=0), (2 * D, D))
    eye = jnp.eye(D, dtype=jnp.float32)
    zeros = jnp.zeros((N_SUB_PAD, D), jnp.float32)
    es = [sc(feats, whl, eye) for sc in _scores_calls]
    outs = [scq(feats, e, ids, zeros) for scq, (e, _) in zip(_sc_calls, es)]
    max_e = jnp.maximum(jnp.max(es[0][1]), jnp.max(es[1][1]))
    for _, mq in es[2:]:
        max_e = jnp.maximum(max_e, jnp.max(mq))
    m = max_e.reshape(1, 1)
    out_pad = _combine([p for p, _ in outs], [z for _, z in outs], m)
    return out_pad[:N_SUB]
